# baseline (speedup 1.0000x reference)
import functools

import jax
import jax.numpy as jnp
from jax import lax
from jax.experimental import pallas as pl
from jax.experimental.pallas import tpu as pltpu
from jax.experimental.pallas import tpu_sc as plsc

B = 1024
D = 256
L = 100000
K = 4096
NEG = 32
CLOSE = 16

NC = 2
NS = 16
NW = NC * NS
BPW = B // NW
NEG_PER_W = B * NEG // NW
NEG_CHUNK = 128
N_NEG_CHUNKS = NEG_PER_W // NEG_CHUNK
GRP = NEG_CHUNK // NEG
NL = D // 16

BLK = 128
NB = B // BLK


def _sc_cluster_body(idx_hbm, labank_hbm, cent_hbm,
                     labels_out, poscent_out,
                     idx_v, labels_v, poscent_v, sem):
    wid = lax.axis_index("s") * NC + lax.axis_index("c")
    base = wid * BPW

    pltpu.sync_copy(idx_hbm.at[wid], idx_v)
    pltpu.async_copy(labank_hbm.at[idx_v], labels_v, sem).wait()
    pltpu.sync_copy(labels_v, labels_out.at[pl.ds(base, BPW)])
    pltpu.async_copy(cent_hbm.at[labels_v], poscent_v, sem).wait()
    pltpu.sync_copy(poscent_v, poscent_out.at[pl.ds(base, BPW)])


@functools.cache
def _make_sc_cluster():
    return pl.kernel(
        _sc_cluster_body,
        out_type=[
            jax.ShapeDtypeStruct((B,), jnp.int32),
            jax.ShapeDtypeStruct((B, D), jnp.float32),
        ],
        mesh=plsc.VectorSubcoreMesh(core_axis_name="c", subcore_axis_name="s"),
        scratch_types=[
            pltpu.VMEM((BPW,), jnp.int32),
            pltpu.VMEM((BPW,), jnp.int32),
            pltpu.VMEM((BPW, D), jnp.float32),
            pltpu.SemaphoreType.DMA,
        ],
    )


def _sc_instance_body(idx_hbm, negidx_hbm, bank_hbm, feat_hbm,
                      ipp_out, inpp_out,
                      idx_v, negidx_v, feat_v, rowbuf_v, negbuf_v, partial_v,
                      sem_a, sem_b):
    wid = lax.axis_index("s") * NC + lax.axis_index("c")
    base = wid * BPW
    nbase = wid * NEG_PER_W

    pltpu.sync_copy(idx_hbm.at[wid], idx_v)
    pltpu.sync_copy(negidx_hbm.at[wid], negidx_v)
    pltpu.sync_copy(feat_hbm.at[pl.ds(base, BPW)], feat_v)

    pltpu.async_copy(bank_hbm.at[idx_v], rowbuf_v, sem_a).wait()
    for b in range(BPW):
        acc = rowbuf_v[b, pl.ds(0, 16)] * feat_v[b, pl.ds(0, 16)]
        for g in range(1, NL):
            acc = acc + (rowbuf_v[b, pl.ds(g * 16, 16)] *
                         feat_v[b, pl.ds(g * 16, 16)])
        partial_v[pl.ds(b * 16, 16)] = acc
    pltpu.sync_copy(partial_v.at[pl.ds(0, BPW * 16)],
                    ipp_out.at[pl.ds(base * 16, BPW * 16)])

    sems = (sem_a, sem_b)
    copies = [pltpu.async_copy(bank_hbm.at[negidx_v.at[0]],
                               negbuf_v.at[0], sem_a)]
    for c in range(N_NEG_CHUNKS):
        if c + 1 < N_NEG_CHUNKS:
            copies.append(
                pltpu.async_copy(bank_hbm.at[negidx_v.at[c + 1]],
                                 negbuf_v.at[(c + 1) % 2],
                                 sems[(c + 1) % 2]))
        copies[c].wait()
        buf = negbuf_v.at[c % 2]
        for g in range(GRP):
            bl = c * GRP + g
            feat_regs = [feat_v[bl, pl.ds(k * 16, 16)] for k in range(NL)]

            def body(r, _, buf=buf, feat_regs=feat_regs, g=g):
                row = g * NEG + r
                acc = buf[row, pl.ds(0, 16)] * feat_regs[0]
                for k in range(1, NL):
                    acc = acc + buf[row, pl.ds(k * 16, 16)] * feat_regs[k]
                partial_v[pl.ds(row * 16, 16)] = acc
                return 0

            lax.fori_loop(0, NEG, body, 0)
        pltpu.sync_copy(
            partial_v,
            inpp_out.at[pl.ds((nbase + c * NEG_CHUNK) * 16, NEG_CHUNK * 16)])


@functools.cache
def _make_sc_instance():
    return pl.kernel(
        _sc_instance_body,
        out_type=[
            jax.ShapeDtypeStruct((B * 16,), jnp.float32),
            jax.ShapeDtypeStruct((B * NEG * 16,), jnp.float32),
        ],
        mesh=plsc.VectorSubcoreMesh(core_axis_name="c", subcore_axis_name="s"),
        scratch_types=[
            pltpu.VMEM((BPW,), jnp.int32),
            pltpu.VMEM((N_NEG_CHUNKS, NEG_CHUNK), jnp.int32),
            pltpu.VMEM((BPW, D), jnp.float32),
            pltpu.VMEM((BPW, D), jnp.float32),
            pltpu.VMEM((2, NEG_CHUNK, D), jnp.float32),
            pltpu.VMEM((NEG_CHUNK * 16,), jnp.float32),
            pltpu.SemaphoreType.DMA,
            pltpu.SemaphoreType.DMA,
        ],
    )


def _tc_cluster_body(feat_ref, poscent_ref, labs_ref, cent_ref,
                     cps_ref, cnsT_ref):
    f = feat_ref[...]
    pc = poscent_ref[...]
    cps_ref[...] = jnp.sum(f * pc, axis=1, keepdims=True)

    cent = cent_ref[...]
    dims = (((1,), (1,)), ((), ()))
    sqcol = jnp.sum(cent * cent, axis=1, keepdims=True)
    cpcT = lax.dot_general(cent, pc, dims,
                           preferred_element_type=jnp.float32)
    simsT = lax.dot_general(cent, f, dims,
                            preferred_element_type=jnp.float32)

    labs = labs_ref[0]
    kio = lax.broadcasted_iota(jnp.int32, (K, BLK), 0)
    big = jnp.float32(3.0e38)
    selfmask = kio == labs
    sq_pos = jnp.min(jnp.where(selfmask, sqcol, big), axis=0,
                     keepdims=True)
    d2 = (sq_pos + sqcol) - 2.0 * cpcT
    dist = jnp.sqrt(jnp.maximum(d2, 0.0))
    dist = jnp.where(selfmask, big, dist)

    key = lax.bitcast_convert_type(dist, jnp.int32)
    hi = lax.shift_right_logical(key, 12)
    lo = jnp.bitwise_or(lax.shift_left(jnp.bitwise_and(key, 0xFFF), 12), kio)
    ibig = jnp.int32(0x7FFFFFFF)
    for j in range(CLOSE):
        m_hi = jnp.min(hi, axis=0, keepdims=True)
        m_lo = jnp.min(jnp.where(hi == m_hi, lo, ibig), axis=0, keepdims=True)
        eqi = (hi == m_hi) & (lo == m_lo)
        cnsT_ref[j:j + 1, :] = jnp.sum(jnp.where(eqi, simsT, 0.0), axis=0,
                                       keepdims=True)
        hi = jnp.where(eqi, ibig, hi)


def _make_tc_cluster(interpret=False):
    return pl.pallas_call(
        _tc_cluster_body,
        grid=(NB,),
        in_specs=[
            pl.BlockSpec((BLK, D), lambda i: (i, 0)),
            pl.BlockSpec((BLK, D), lambda i: (i, 0)),
            pl.BlockSpec((1, 1, BLK), lambda i: (i, 0, 0)),
            pl.BlockSpec((K, D), lambda i: (0, 0)),
        ],
        out_specs=[
            pl.BlockSpec((BLK, 1), lambda i: (i, 0)),
            pl.BlockSpec((CLOSE, BLK), lambda i: (0, i)),
        ],
        out_shape=[
            jax.ShapeDtypeStruct((B, 1), jnp.float32),
            jax.ShapeDtypeStruct((CLOSE, B), jnp.float32),
        ],
        interpret=interpret,
    )


def _tc_reduce_body(ipp_ref, inpp_ref, ips_ref, ins_ref):
    ips_ref[...] = jnp.sum(ipp_ref[...], axis=1, keepdims=True)
    ins_ref[...] = jnp.sum(inpp_ref[...], axis=2)


def _make_tc_reduce(interpret=False):
    return pl.pallas_call(
        _tc_reduce_body,
        grid=(NB,),
        in_specs=[
            pl.BlockSpec((BLK, 16), lambda i: (i, 0)),
            pl.BlockSpec((BLK, NEG, 16), lambda i: (i, 0, 0)),
        ],
        out_specs=[
            pl.BlockSpec((BLK, 1), lambda i: (i, 0)),
            pl.BlockSpec((BLK, NEG), lambda i: (i, 0)),
        ],
        out_shape=[
            jax.ShapeDtypeStruct((B, 1), jnp.float32),
            jax.ShapeDtypeStruct((B, NEG), jnp.float32),
        ],
        interpret=interpret,
    )


_tc_cluster = _make_tc_cluster()
_tc_reduce = _make_tc_reduce()


@jax.jit
def kernel(feature, idx, neg_indices, feature_bank, label_bank, centroids):
    idx = idx.astype(jnp.int32)
    idx2 = idx.reshape(NW, BPW)
    neg3 = neg_indices.reshape(NW, N_NEG_CHUNKS, NEG_CHUNK)
    labels, poscent = _make_sc_cluster()(idx2, label_bank, centroids)
    ipp, inpp = _make_sc_instance()(idx2, neg3, feature_bank, feature)
    cps, cnsT = _tc_cluster(feature, poscent,
                            labels.reshape(NB, 1, BLK), centroids)
    ips, ins = _tc_reduce(ipp.reshape(B, 16), inpp.reshape(B, NEG, 16))
    return ips, ins, cps, cnsT.T


# speedup vs baseline: 1.1201x; 1.1201x over previous
import functools

import jax
import jax.numpy as jnp
from jax import lax
from jax.experimental import pallas as pl
from jax.experimental.pallas import tpu as pltpu
from jax.experimental.pallas import tpu_sc as plsc

B = 1024
D = 256
L = 100000
K = 4096
NEG = 32
CLOSE = 16

NC = 2
NS = 16
NW = NC * NS
BPW = B // NW
NEG_PER_W = B * NEG // NW
NEG_CHUNK = 128
N_NEG_CHUNKS = NEG_PER_W // NEG_CHUNK

BLK = 128
NB = B // BLK


def _sc_gather_body(idx_hbm, negidx_hbm, bank_hbm, labank_hbm, cent_hbm,
                    labels_out, poscent_out, inspos_out, insneg_out,
                    idx_v, labels_v, poscent_v, inspos_v,
                    negidx_v, negbuf_v, sem_a, sem_b):
    wid = lax.axis_index("s") * NC + lax.axis_index("c")
    base = wid * BPW

    pltpu.sync_copy(idx_hbm.at[wid], idx_v)

    pltpu.async_copy(labank_hbm.at[idx_v], labels_v, sem_a).wait()
    pltpu.sync_copy(labels_v, labels_out.at[pl.ds(base, BPW)])
    pltpu.async_copy(cent_hbm.at[labels_v], poscent_v, sem_a).wait()
    pltpu.sync_copy(poscent_v, poscent_out.at[pl.ds(base, BPW)])

    pltpu.async_copy(bank_hbm.at[idx_v], inspos_v, sem_a).wait()
    pltpu.sync_copy(inspos_v, inspos_out.at[pl.ds(base, BPW)])

    nbase = wid * NEG_PER_W
    pltpu.sync_copy(negidx_hbm.at[wid], negidx_v)
    sems = (sem_a, sem_b)
    copies = [pltpu.async_copy(bank_hbm.at[negidx_v.at[0]],
                               negbuf_v.at[0], sem_a)]
    for c in range(N_NEG_CHUNKS):
        if c + 1 < N_NEG_CHUNKS:
            copies.append(
                pltpu.async_copy(bank_hbm.at[negidx_v.at[c + 1]],
                                 negbuf_v.at[(c + 1) % 2],
                                 sems[(c + 1) % 2]))
        copies[c].wait()
        pltpu.sync_copy(
            negbuf_v.at[c % 2],
            insneg_out.at[pl.ds(nbase + c * NEG_CHUNK, NEG_CHUNK)])


@functools.cache
def _make_sc_gather():
    return pl.kernel(
        _sc_gather_body,
        out_type=[
            jax.ShapeDtypeStruct((B,), jnp.int32),
            jax.ShapeDtypeStruct((B, D), jnp.float32),
            jax.ShapeDtypeStruct((B, D), jnp.float32),
            jax.ShapeDtypeStruct((B * NEG, D), jnp.float32),
        ],
        mesh=plsc.VectorSubcoreMesh(core_axis_name="c", subcore_axis_name="s"),
        scratch_types=[
            pltpu.VMEM((BPW,), jnp.int32),
            pltpu.VMEM((BPW,), jnp.int32),
            pltpu.VMEM((BPW, D), jnp.float32),
            pltpu.VMEM((BPW, D), jnp.float32),
            pltpu.VMEM((N_NEG_CHUNKS, NEG_CHUNK), jnp.int32),
            pltpu.VMEM((2, NEG_CHUNK, D), jnp.float32),
            pltpu.SemaphoreType.DMA,
            pltpu.SemaphoreType.DMA,
        ],
    )


def _tc_body(feat_ref, poscent_ref, inspos_ref, insneg_ref, labs_ref,
             cent_ref, ips_ref, ins_ref, cps_ref, cnsT_ref):
    f = feat_ref[...]
    pc = poscent_ref[...]

    ips_ref[...] = jnp.sum(f * inspos_ref[...], axis=1, keepdims=True)
    cps_ref[...] = jnp.sum(f * pc, axis=1, keepdims=True)
    ins_ref[...] = jnp.sum(insneg_ref[...] * f[:, None, :], axis=2)

    cent = cent_ref[...]
    dims = (((1,), (1,)), ((), ()))
    sqcol = jnp.sum(cent * cent, axis=1, keepdims=True)
    cpcT = lax.dot_general(cent, pc, dims,
                           preferred_element_type=jnp.float32)
    simsT = lax.dot_general(cent, f, dims,
                            preferred_element_type=jnp.float32)

    labs = labs_ref[0]
    kio = lax.broadcasted_iota(jnp.int32, (K, BLK), 0)
    big = jnp.float32(3.0e38)
    selfmask = kio == labs
    sq_pos = jnp.min(jnp.where(selfmask, sqcol, big), axis=0,
                     keepdims=True)
    d2 = (sq_pos + sqcol) - 2.0 * cpcT
    dist = jnp.sqrt(jnp.maximum(d2, 0.0))
    dist = jnp.where(selfmask, big, dist)

    key = lax.bitcast_convert_type(dist, jnp.int32)
    ibig = jnp.int32(0x7FFFFFFF)
    for j in range(CLOSE):
        m = jnp.min(key, axis=0, keepdims=True)
        hit = key == m
        idxv = jnp.min(jnp.where(hit, kio, K), axis=0, keepdims=True)
        eqi = hit & (kio == idxv)
        cnsT_ref[j:j + 1, :] = jnp.sum(jnp.where(eqi, simsT, 0.0), axis=0,
                                       keepdims=True)
        key = jnp.where(eqi, ibig, key)


def _make_tc(interpret=False):
    return pl.pallas_call(
        _tc_body,
        grid=(NB,),
        in_specs=[
            pl.BlockSpec((BLK, D), lambda i: (i, 0)),
            pl.BlockSpec((BLK, D), lambda i: (i, 0)),
            pl.BlockSpec((BLK, D), lambda i: (i, 0)),
            pl.BlockSpec((BLK, NEG, D), lambda i: (i, 0, 0)),
            pl.BlockSpec((1, 1, BLK), lambda i: (i, 0, 0)),
            pl.BlockSpec((K, D), lambda i: (0, 0)),
        ],
        out_specs=[
            pl.BlockSpec((BLK, 1), lambda i: (i, 0)),
            pl.BlockSpec((BLK, NEG), lambda i: (i, 0)),
            pl.BlockSpec((BLK, 1), lambda i: (i, 0)),
            pl.BlockSpec((CLOSE, BLK), lambda i: (0, i)),
        ],
        out_shape=[
            jax.ShapeDtypeStruct((B, 1), jnp.float32),
            jax.ShapeDtypeStruct((B, NEG), jnp.float32),
            jax.ShapeDtypeStruct((B, 1), jnp.float32),
            jax.ShapeDtypeStruct((CLOSE, B), jnp.float32),
        ],
        interpret=interpret,
    )


_tc_call = _make_tc()


@jax.jit
def kernel(feature, idx, neg_indices, feature_bank, label_bank, centroids):
    idx = idx.astype(jnp.int32)
    labels, poscent, inspos, insneg = _make_sc_gather()(
        idx.reshape(NW, BPW),
        neg_indices.reshape(NW, N_NEG_CHUNKS, NEG_CHUNK),
        feature_bank, label_bank, centroids)
    ips, ins, cps, cnsT = _tc_call(feature, poscent, inspos,
                                   insneg.reshape(B, NEG, D),
                                   labels.reshape(NB, 1, BLK), centroids)
    return ips, ins, cps, cnsT.T
